# Initial kernel scaffold; baseline (speedup 1.0000x reference)
import jax
import jax.numpy as jnp
from jax.experimental import pallas as pl


def kernel(x, tables):
    raise NotImplementedError("write your pallas kernel here")


# speedup vs baseline: 1.1452x; 1.1452x over previous
import functools

import jax
import jax.numpy as jnp
from jax import lax
from jax.experimental import pallas as pl
from jax.experimental.pallas import tpu as pltpu
from jax.experimental.pallas import tpu_sc as plsc

N_CAT = 26
VOCAB = 100000
D_MODEL = 32
BATCH = 16384

_NC = 2
_NS = 16
_NW = _NC * _NS

_ROWS = BATCH * N_CAT
_PER_W = _ROWS // _NW
_CHUNK = 1664
_NCHUNK = _PER_W // _CHUNK
_LANES = 16
_VECS = _CHUNK // _LANES


@functools.partial(
    pl.kernel,
    out_type=jax.ShapeDtypeStruct((_ROWS, D_MODEL), jnp.float32),
    mesh=plsc.VectorSubcoreMesh(core_axis_name="c", subcore_axis_name="s"),
    compiler_params=pltpu.CompilerParams(use_tc_tiling_on_sc=False),
    scratch_types=[
        pltpu.VMEM((_CHUNK,), jnp.int32),
        pltpu.VMEM((_CHUNK,), jnp.int32),
        pltpu.VMEM((_CHUNK, D_MODEL), jnp.float32),
        pltpu.SemaphoreType.DMA,
    ],
)
def _gather_kernel(x_hbm, tab_hbm, out_hbm, off_v, idx_v, rows_v, sem):
    wid = lax.axis_index("s") * _NC + lax.axis_index("c")
    base = wid * _PER_W

    lane = lax.iota(jnp.int32, _LANES)

    def _mk_off(j, _):
        col = (j * _LANES + lane) % N_CAT
        off_v[pl.ds(j * _LANES, _LANES)] = col * VOCAB
        return 0

    lax.fori_loop(0, _VECS, _mk_off, 0)

    def _chunk(k, _):
        c0 = base + k * _CHUNK
        pltpu.sync_copy(x_hbm.at[pl.ds(c0, _CHUNK)], idx_v)

        def _add_off(j, _):
            s = pl.ds(j * _LANES, _LANES)
            idx_v[s] = idx_v[s] + off_v[s]
            return 0

        lax.fori_loop(0, _VECS, _add_off, 0)
        pltpu.async_copy(tab_hbm.at[idx_v], rows_v, sem).wait()
        pltpu.sync_copy(rows_v, out_hbm.at[pl.ds(c0, _CHUNK)])
        return 0

    lax.fori_loop(0, _NCHUNK, _chunk, 0)


def kernel(x, tables):
    x_flat = x.astype(jnp.int32).reshape(_ROWS)
    tab_flat = tables.reshape(N_CAT * VOCAB, D_MODEL)
    out = _gather_kernel(x_flat, tab_flat)
    return out.reshape(BATCH, N_CAT, D_MODEL)


# speedup vs baseline: 1.1507x; 1.0048x over previous
import functools

import jax
import jax.numpy as jnp
from jax import lax
from jax.experimental import pallas as pl
from jax.experimental.pallas import tpu as pltpu
from jax.experimental.pallas import tpu_sc as plsc

N_CAT = 26
VOCAB = 100000
D_MODEL = 32
BATCH = 16384

_NC = 2
_NS = 16
_NW = _NC * _NS

_ROWS = BATCH * N_CAT
_PER_W = _ROWS // _NW
_CHUNK = 1664
_NCHUNK = _PER_W // _CHUNK
_LANES = 16
_VECS = _CHUNK // _LANES


@functools.partial(
    pl.kernel,
    out_type=jax.ShapeDtypeStruct((_ROWS, D_MODEL), jnp.float32),
    mesh=plsc.VectorSubcoreMesh(core_axis_name="c", subcore_axis_name="s"),
    compiler_params=pltpu.CompilerParams(use_tc_tiling_on_sc=False),
    scratch_types=[
        pltpu.VMEM((_PER_W,), jnp.int32),
        pltpu.VMEM((2, _CHUNK, D_MODEL), jnp.float32),
        pltpu.SemaphoreType.DMA,
        pltpu.SemaphoreType.DMA,
    ],
)
def _gather_kernel(x_hbm, tab_hbm, out_hbm, idx_v, rows_v, gsem, wsem):
    wid = lax.axis_index("s") * _NC + lax.axis_index("c")
    base = wid * _PER_W

    pltpu.sync_copy(x_hbm.at[pl.ds(base, _PER_W)], idx_v)

    lane = lax.iota(jnp.int32, _LANES)

    def _add_off(j, _):
        s = pl.ds(j * _LANES, _LANES)
        idx_v[s] = idx_v[s] + ((j * _LANES + lane) % N_CAT) * VOCAB
        return 0

    def _offsets_for_chunk(k):
        lax.fori_loop(k * _VECS, (k + 1) * _VECS, _add_off, 0)

    def _start_gather(k):
        return pltpu.async_copy(
            tab_hbm.at[idx_v.at[pl.ds(k * _CHUNK, _CHUNK)]],
            rows_v.at[k % 2], gsem)

    def _start_writeback(k):
        return pltpu.async_copy(
            rows_v.at[k % 2], out_hbm.at[pl.ds(base + k * _CHUNK, _CHUNK)],
            wsem)

    _offsets_for_chunk(0)
    gathers = {0: _start_gather(0)}
    writebacks = {}
    for k in range(1, _NCHUNK):
        _offsets_for_chunk(k)
        gathers[k - 1].wait()
        writebacks[k - 1] = _start_writeback(k - 1)
        if k >= 2:
            writebacks[k - 2].wait()
        gathers[k] = _start_gather(k)
    gathers[_NCHUNK - 1].wait()
    writebacks[_NCHUNK - 1] = _start_writeback(_NCHUNK - 1)
    writebacks[_NCHUNK - 2].wait()
    writebacks[_NCHUNK - 1].wait()


def kernel(x, tables):
    x_flat = x.astype(jnp.int32).reshape(_ROWS)
    tab_flat = tables.reshape(N_CAT * VOCAB, D_MODEL)
    out = _gather_kernel(x_flat, tab_flat)
    return out.reshape(BATCH, N_CAT, D_MODEL)


# speedup vs baseline: 1.1535x; 1.0023x over previous
import functools

import jax
import jax.numpy as jnp
from jax import lax
from jax.experimental import pallas as pl
from jax.experimental.pallas import tpu as pltpu
from jax.experimental.pallas import tpu_sc as plsc

N_CAT = 26
VOCAB = 100000
D_MODEL = 32
BATCH = 16384

_NC = 2
_NS = 16
_NW = _NC * _NS

_ROWS = BATCH * N_CAT
_PER_W = _ROWS // _NW
_CHUNK = 832
_NCHUNK = _PER_W // _CHUNK
_NBUF = 4
_INFL = 3
_LANES = 16
_VECS = _CHUNK // _LANES


@functools.partial(
    pl.kernel,
    out_type=jax.ShapeDtypeStruct((_ROWS, D_MODEL), jnp.float32),
    mesh=plsc.VectorSubcoreMesh(core_axis_name="c", subcore_axis_name="s"),
    compiler_params=pltpu.CompilerParams(use_tc_tiling_on_sc=False),
    scratch_types=[
        pltpu.VMEM((_PER_W,), jnp.int32),
        pltpu.VMEM((_NBUF, _CHUNK, D_MODEL), jnp.float32),
        pltpu.SemaphoreType.DMA,
        pltpu.SemaphoreType.DMA,
    ],
)
def _gather_kernel(x_hbm, tab_hbm, out_hbm, idx_v, rows_v, gsem, wsem):
    wid = lax.axis_index("s") * _NC + lax.axis_index("c")
    base = wid * _PER_W

    pltpu.sync_copy(x_hbm.at[pl.ds(base, _PER_W)], idx_v)

    lane = lax.iota(jnp.int32, _LANES)

    def _add_off(j, _):
        s = pl.ds(j * _LANES, _LANES)
        idx_v[s] = idx_v[s] + ((j * _LANES + lane) % N_CAT) * VOCAB
        return 0

    lax.fori_loop(0, _PER_W // _LANES, _add_off, 0)

    def _start_gather(k):
        return pltpu.async_copy(
            tab_hbm.at[idx_v.at[pl.ds(k * _CHUNK, _CHUNK)]],
            rows_v.at[k % _NBUF], gsem)

    def _start_writeback(k):
        return pltpu.async_copy(
            rows_v.at[k % _NBUF], out_hbm.at[pl.ds(base + k * _CHUNK, _CHUNK)],
            wsem)

    gathers, writebacks = {}, {}
    for t in range(_NCHUNK + _INFL):
        if t < _NCHUNK:
            if t >= _NBUF:
                writebacks[t - _NBUF].wait()
            gathers[t] = _start_gather(t)
        j = t - _INFL
        if 0 <= j < _NCHUNK:
            gathers[j].wait()
            writebacks[j] = _start_writeback(j)
    for j in range(_NCHUNK - _NBUF, _NCHUNK):
        writebacks[j].wait()


def kernel(x, tables):
    x_flat = x.astype(jnp.int32).reshape(_ROWS)
    tab_flat = tables.reshape(N_CAT * VOCAB, D_MODEL)
    out = _gather_kernel(x_flat, tab_flat)
    return out.reshape(BATCH, N_CAT, D_MODEL)


# speedup vs baseline: 4.3824x; 3.7994x over previous
import functools

import jax
import jax.numpy as jnp
from jax import lax
from jax.experimental import pallas as pl
from jax.experimental.pallas import tpu as pltpu
from jax.experimental.pallas import tpu_sc as plsc

N_CAT = 26
VOCAB = 100000
D_MODEL = 32
BATCH = 16384

_NC = 2
_NS = 16
_NW = _NC * _NS

_HALF = BATCH // 2
_LANES = 16


@functools.partial(
    pl.kernel,
    out_type=jax.ShapeDtypeStruct((N_CAT, D_MODEL, BATCH), jnp.float32),
    mesh=plsc.VectorSubcoreMesh(core_axis_name="c", subcore_axis_name="s"),
    compiler_params=pltpu.CompilerParams(needs_layout_passes=False),
    scratch_types=[
        pltpu.VMEM((VOCAB,), jnp.float32),
        pltpu.VMEM((_HALF,), jnp.int32),
        pltpu.VMEM((_HALF,), jnp.float32),
    ],
)
def _gather_kernel(x_hbm, tab_hbm, out_hbm, row_v, idx_v, out_v):
    d = lax.axis_index("s") * _NC + lax.axis_index("c")

    def _per_cat(c, _):
        pltpu.sync_copy(tab_hbm.at[c, d, :], row_v)
        for h in range(2):
            pltpu.sync_copy(x_hbm.at[c, pl.ds(h * _HALF, _HALF)], idx_v)

            def _g(j, _):
                s = pl.ds(j * _LANES, _LANES)
                out_v[s] = plsc.load_gather(row_v, [idx_v[s]])
                return 0

            lax.fori_loop(0, _HALF // _LANES, _g, 0)
            pltpu.sync_copy(out_v, out_hbm.at[c, d, pl.ds(h * _HALF, _HALF)])
        return 0

    lax.fori_loop(0, N_CAT, _per_cat, 0)


def kernel(x, tables):
    x_t = x.T.astype(jnp.int32)
    tab_t = jnp.transpose(tables, (0, 2, 1))
    out_t = _gather_kernel(x_t, tab_t)
    return jnp.transpose(out_t, (2, 0, 1))


# speedup vs baseline: 5.7046x; 1.3017x over previous
import functools

import jax
import jax.numpy as jnp
from jax import lax
from jax.experimental import pallas as pl
from jax.experimental.pallas import tpu as pltpu
from jax.experimental.pallas import tpu_sc as plsc

N_CAT = 26
VOCAB = 100000
D_MODEL = 32
BATCH = 16384

_NC = 2
_NS = 16
_NW = _NC * _NS

_QB = BATCH // 4
_LANES = 16


@functools.partial(
    pl.kernel,
    out_type=jax.ShapeDtypeStruct((N_CAT, D_MODEL, BATCH), jnp.float32),
    mesh=plsc.VectorSubcoreMesh(core_axis_name="c", subcore_axis_name="s"),
    compiler_params=pltpu.CompilerParams(needs_layout_passes=False),
    scratch_types=[
        pltpu.VMEM((VOCAB,), jnp.float32),
        pltpu.VMEM((2, _QB), jnp.int32),
        pltpu.VMEM((2, _QB), jnp.float32),
        pltpu.SemaphoreType.DMA,
        pltpu.SemaphoreType.DMA,
        pltpu.SemaphoreType.DMA,
    ],
)
def _gather_kernel(x_hbm, tab_hbm, out_hbm, row_v, idx_v, out_v, rsem, isem, osem):
    d = lax.axis_index("s") * _NC + lax.axis_index("c")
    _UNROLL = 8

    def _gather_chunk(b):
        def _g(j, _):
            for u in range(_UNROLL):
                s = pl.ds((j * _UNROLL + u) * _LANES, _LANES)
                out_v[b, s] = plsc.load_gather(row_v, [idx_v[b, s]])
            return 0

        lax.fori_loop(0, _QB // (_LANES * _UNROLL), _g, 0)

    def _per_cat(c, _):
        r0 = pltpu.async_copy(tab_hbm.at[c, d, :], row_v, rsem)
        i_prev = pltpu.async_copy(x_hbm.at[c, pl.ds(0, _QB)], idx_v.at[0], isem)
        i_prev.wait()
        r0.wait()
        outs = {}
        for q in range(4):
            b = q % 2
            if q < 3:
                i_next = pltpu.async_copy(
                    x_hbm.at[c, pl.ds((q + 1) * _QB, _QB)],
                    idx_v.at[(q + 1) % 2], isem)
            if q >= 2:
                outs[q - 2].wait()
            _gather_chunk(b)
            outs[q] = pltpu.async_copy(
                out_v.at[b], out_hbm.at[c, d, pl.ds(q * _QB, _QB)], osem)
            if q < 3:
                i_next.wait()
        outs[2].wait()
        outs[3].wait()
        return 0

    lax.fori_loop(0, N_CAT, _per_cat, 0)


def kernel(x, tables):
    x_t = x.T.astype(jnp.int32)
    tab_t = jnp.transpose(tables, (0, 2, 1))
    out_t = _gather_kernel(x_t, tab_t)
    return jnp.transpose(out_t, (2, 0, 1))
